# baseline (speedup 1.0000x reference)
import functools

import jax
import jax.numpy as jnp
from jax import lax
from jax.experimental import pallas as pl
from jax.experimental.pallas import tpu as pltpu
from jax.experimental.pallas import tpu_sc as plsc

B = 16384
S = 50
D = 64
ROW = S * D
NC = 2
NS = 16
NW = NC * NS
RPW = B // NW
CH = 8
NCH = RPW // CH
GSZ = 128
NG = RPW // GSZ
EPS = 0.001
L = 16


def _normalize_chunk(xb, mrows, srows, chunk):
    for i in range(CH):
        row = chunk * CH + i
        ms = [mrows[row, pl.ds(j * L, L)] for j in range(D // L)]
        rs = [srows[row, pl.ds(j * L, L)] for j in range(D // L)]

        def body(k, _, ms=ms, rs=rs, i=i):
            for j in range(D // L):
                off = k * D + j * L
                xb[i, pl.ds(off, L)] = (xb[i, pl.ds(off, L)] - ms[j]) * rs[j]
            return 0

        lax.fori_loop(0, S, body, 0)


def _scbnorm_body(x_hbm, cid_hbm, mean_hbm, std_hbm, out_hbm,
                  idx0, idx1, idx2, idx3, mrows, srows, xb0, xb1,
                  sem_g, si0, si1, so0, so1):
    wid = lax.axis_index("s") * NC + lax.axis_index("c")
    base = wid * RPW
    idxs = [idx0, idx1, idx2, idx3]

    for g in range(NG):
        pltpu.sync_copy(cid_hbm.at[pl.ds(base + g * GSZ, GSZ)], idxs[g])
    gathers = []
    for g in range(NG):
        dst_m = mrows.at[pl.ds(g * GSZ, GSZ)]
        dst_s = srows.at[pl.ds(g * GSZ, GSZ)]
        gathers.append(pltpu.async_copy(mean_hbm.at[idxs[g]], dst_m, sem_g))
        gathers.append(pltpu.async_copy(std_hbm.at[idxs[g]], dst_s, sem_g))
    for cp in gathers:
        cp.wait()

    def rinv_body(i, _):
        for j in range(D // L):
            v = srows[i, pl.ds(j * L, L)]
            srows[i, pl.ds(j * L, L)] = 1.0 / (jnp.exp(v) + EPS)
        return 0

    lax.fori_loop(0, RPW, rinv_body, 0)

    def in_cp(xb, sem, chunk):
        return pltpu.make_async_copy(
            x_hbm.at[pl.ds(base + chunk * CH, CH)], xb, sem)

    def out_cp(xb, sem, chunk):
        return pltpu.make_async_copy(
            xb, out_hbm.at[pl.ds(base + chunk * CH, CH)], sem)

    in_cp(xb0, si0, 0).start()

    def pair_body(g, _):
        c0 = 2 * g
        @pl.when(g > 0)
        def _():
            out_cp(xb1, so1, c0 - 1).wait()
        in_cp(xb1, si1, c0 + 1).start()

        in_cp(xb0, si0, c0).wait()
        _normalize_chunk(xb0, mrows, srows, c0)
        out_cp(xb0, so0, c0).start()

        @pl.when(g < NCH // 2 - 1)
        def _():
            out_cp(xb0, so0, c0).wait()
            in_cp(xb0, si0, c0 + 2).start()

        in_cp(xb1, si1, c0 + 1).wait()
        _normalize_chunk(xb1, mrows, srows, c0 + 1)
        out_cp(xb1, so1, c0 + 1).start()
        return 0

    lax.fori_loop(0, NCH // 2, pair_body, 0)
    out_cp(xb0, so0, NCH - 2).wait()
    out_cp(xb1, so1, NCH - 1).wait()


@jax.jit
def kernel(x, cluster_id, initial_mean, initial_std):
    x2d = x.reshape(B, ROW)
    cid = cluster_id.reshape(B)

    mesh = plsc.VectorSubcoreMesh(core_axis_name="c", subcore_axis_name="s")
    run = functools.partial(
        pl.kernel,
        out_type=jax.ShapeDtypeStruct((B, ROW), jnp.float32),
        mesh=mesh,
        compiler_params=pltpu.CompilerParams(use_tc_tiling_on_sc=False),
        scratch_types=[
            pltpu.VMEM((GSZ,), jnp.int32),
            pltpu.VMEM((GSZ,), jnp.int32),
            pltpu.VMEM((GSZ,), jnp.int32),
            pltpu.VMEM((GSZ,), jnp.int32),
            pltpu.VMEM((RPW, D), jnp.float32),
            pltpu.VMEM((RPW, D), jnp.float32),
            pltpu.VMEM((CH, ROW), jnp.float32),
            pltpu.VMEM((CH, ROW), jnp.float32),
            pltpu.SemaphoreType.DMA,
            pltpu.SemaphoreType.DMA,
            pltpu.SemaphoreType.DMA,
            pltpu.SemaphoreType.DMA,
            pltpu.SemaphoreType.DMA,
        ],
    )(_scbnorm_body)
    out = run(x2d, cid, initial_mean, initial_std)
    return out.reshape(B, S, D)


# speedup vs baseline: 1.0732x; 1.0732x over previous
import functools

import jax
import jax.numpy as jnp
from jax import lax
from jax.experimental import pallas as pl
from jax.experimental.pallas import tpu as pltpu
from jax.experimental.pallas import tpu_sc as plsc

B = 16384
S = 50
D = 64
ROW = S * D
NC = 2
NS = 16
NW = NC * NS
RPW = B // NW
CH = 8
NCH = RPW // CH
GSZ = 128
NG = RPW // GSZ
EPS = 0.001
L = 16


def _normalize_chunk(xb, mrows, srows, chunk):
    for i in range(CH):
        row = chunk * CH + i
        ms = [mrows[row, pl.ds(j * L, L)] for j in range(D // L)]
        rs = [srows[row, pl.ds(j * L, L)] for j in range(D // L)]

        @plsc.parallel_loop(0, S, unroll=5)
        def body(k, ms=ms, rs=rs, i=i):
            for j in range(D // L):
                off = k * D + j * L
                xb[i, pl.ds(off, L)] = (xb[i, pl.ds(off, L)] - ms[j]) * rs[j]


def _scbnorm_body(x_hbm, cid_hbm, mean_hbm, std_hbm, out_hbm,
                  idx0, idx1, idx2, idx3, mrows, srows, xb0, xb1,
                  sem_g, si0, si1, so0, so1):
    wid = lax.axis_index("s") * NC + lax.axis_index("c")
    base = wid * RPW
    idxs = [idx0, idx1, idx2, idx3]

    for g in range(NG):
        pltpu.sync_copy(cid_hbm.at[pl.ds(base + g * GSZ, GSZ)], idxs[g])
    gathers = []
    for g in range(NG):
        dst_m = mrows.at[pl.ds(g * GSZ, GSZ)]
        dst_s = srows.at[pl.ds(g * GSZ, GSZ)]
        gathers.append(pltpu.async_copy(mean_hbm.at[idxs[g]], dst_m, sem_g))
        gathers.append(pltpu.async_copy(std_hbm.at[idxs[g]], dst_s, sem_g))
    for cp in gathers:
        cp.wait()

    def rinv_body(i, _):
        for j in range(D // L):
            v = srows[i, pl.ds(j * L, L)]
            srows[i, pl.ds(j * L, L)] = 1.0 / (jnp.exp(v) + EPS)
        return 0

    lax.fori_loop(0, RPW, rinv_body, 0)

    def in_cp(xb, sem, chunk):
        return pltpu.make_async_copy(
            x_hbm.at[pl.ds(base + chunk * CH, CH)], xb, sem)

    def out_cp(xb, sem, chunk):
        return pltpu.make_async_copy(
            xb, out_hbm.at[pl.ds(base + chunk * CH, CH)], sem)

    in_cp(xb0, si0, 0).start()

    def pair_body(g, _):
        c0 = 2 * g
        @pl.when(g > 0)
        def _():
            out_cp(xb1, so1, c0 - 1).wait()
        in_cp(xb1, si1, c0 + 1).start()

        in_cp(xb0, si0, c0).wait()
        _normalize_chunk(xb0, mrows, srows, c0)
        out_cp(xb0, so0, c0).start()

        @pl.when(g < NCH // 2 - 1)
        def _():
            out_cp(xb0, so0, c0).wait()
            in_cp(xb0, si0, c0 + 2).start()

        in_cp(xb1, si1, c0 + 1).wait()
        _normalize_chunk(xb1, mrows, srows, c0 + 1)
        out_cp(xb1, so1, c0 + 1).start()
        return 0

    lax.fori_loop(0, NCH // 2, pair_body, 0)
    out_cp(xb0, so0, NCH - 2).wait()
    out_cp(xb1, so1, NCH - 1).wait()


@jax.jit
def kernel(x, cluster_id, initial_mean, initial_std):
    x2d = x.reshape(B, ROW)
    cid = cluster_id.reshape(B)

    mesh = plsc.VectorSubcoreMesh(core_axis_name="c", subcore_axis_name="s")
    run = functools.partial(
        pl.kernel,
        out_type=jax.ShapeDtypeStruct((B, ROW), jnp.float32),
        mesh=mesh,
        compiler_params=pltpu.CompilerParams(use_tc_tiling_on_sc=False),
        scratch_types=[
            pltpu.VMEM((GSZ,), jnp.int32),
            pltpu.VMEM((GSZ,), jnp.int32),
            pltpu.VMEM((GSZ,), jnp.int32),
            pltpu.VMEM((GSZ,), jnp.int32),
            pltpu.VMEM((RPW, D), jnp.float32),
            pltpu.VMEM((RPW, D), jnp.float32),
            pltpu.VMEM((CH, ROW), jnp.float32),
            pltpu.VMEM((CH, ROW), jnp.float32),
            pltpu.SemaphoreType.DMA,
            pltpu.SemaphoreType.DMA,
            pltpu.SemaphoreType.DMA,
            pltpu.SemaphoreType.DMA,
            pltpu.SemaphoreType.DMA,
        ],
    )(_scbnorm_body)
    out = run(x2d, cid, initial_mean, initial_std)
    return out.reshape(B, S, D)


# speedup vs baseline: 1.3657x; 1.2725x over previous
import functools

import jax
import jax.numpy as jnp
from jax import lax
from jax.experimental import pallas as pl
from jax.experimental.pallas import tpu as pltpu
from jax.experimental.pallas import tpu_sc as plsc

B = 16384
S = 50
D = 64
NC = 2
NS = 16
NW = NC * NS
RPW = B // NW
GSZ = 128
NG = RPW // GSZ
EPS = 0.001
L = 16
BB = 256


def _gather_body(cid_hbm, mean_hbm, std_hbm, m_hbm, r_hbm,
                 idx0, idx1, idx2, idx3, mrows, srows, sem_g, sem_o):
    wid = lax.axis_index("s") * NC + lax.axis_index("c")
    base = wid * RPW
    idxs = [idx0, idx1, idx2, idx3]

    for g in range(NG):
        pltpu.sync_copy(cid_hbm.at[pl.ds(base + g * GSZ, GSZ)], idxs[g])
    gathers = []
    for g in range(NG):
        dst_m = mrows.at[pl.ds(g * GSZ, GSZ)]
        dst_s = srows.at[pl.ds(g * GSZ, GSZ)]
        gathers.append(pltpu.async_copy(mean_hbm.at[idxs[g]], dst_m, sem_g))
        gathers.append(pltpu.async_copy(std_hbm.at[idxs[g]], dst_s, sem_g))
    for cp in gathers:
        cp.wait()

    out_m = pltpu.async_copy(mrows, m_hbm.at[pl.ds(base, RPW)], sem_o)

    @plsc.parallel_loop(0, RPW)
    def rinv_body(i):
        for j in range(D // L):
            v = srows[i, pl.ds(j * L, L)]
            srows[i, pl.ds(j * L, L)] = 1.0 / (jnp.exp(v) + EPS)

    out_r = pltpu.async_copy(srows, r_hbm.at[pl.ds(base, RPW)], sem_o)
    out_m.wait()
    out_r.wait()


def _sc_gather(cid, initial_mean, initial_std):
    mesh = plsc.VectorSubcoreMesh(core_axis_name="c", subcore_axis_name="s")
    run = functools.partial(
        pl.kernel,
        out_type=(
            jax.ShapeDtypeStruct((B, D), jnp.float32),
            jax.ShapeDtypeStruct((B, D), jnp.float32),
        ),
        mesh=mesh,
        compiler_params=pltpu.CompilerParams(use_tc_tiling_on_sc=False),
        scratch_types=[
            pltpu.VMEM((GSZ,), jnp.int32),
            pltpu.VMEM((GSZ,), jnp.int32),
            pltpu.VMEM((GSZ,), jnp.int32),
            pltpu.VMEM((GSZ,), jnp.int32),
            pltpu.VMEM((RPW, D), jnp.float32),
            pltpu.VMEM((RPW, D), jnp.float32),
            pltpu.SemaphoreType.DMA,
            pltpu.SemaphoreType.DMA,
        ],
    )(_gather_body)
    return run(cid, initial_mean, initial_std)


def _norm_body(x_ref, m_ref, r_ref, o_ref):
    m = jnp.concatenate([m_ref[...], m_ref[...]], axis=-1)
    r = jnp.concatenate([r_ref[...], r_ref[...]], axis=-1)
    o_ref[...] = (x_ref[...] - m[:, None, :]) * r[:, None, :]


def _tc_normalize(x3, m, r):
    return pl.pallas_call(
        _norm_body,
        out_shape=jax.ShapeDtypeStruct((B, S // 2, 2 * D), jnp.float32),
        grid=(B // BB,),
        in_specs=[
            pl.BlockSpec((BB, S // 2, 2 * D), lambda i: (i, 0, 0)),
            pl.BlockSpec((BB, D), lambda i: (i, 0)),
            pl.BlockSpec((BB, D), lambda i: (i, 0)),
        ],
        out_specs=pl.BlockSpec((BB, S // 2, 2 * D), lambda i: (i, 0, 0)),
    )(x3, m, r)


@jax.jit
def kernel(x, cluster_id, initial_mean, initial_std):
    cid = cluster_id.reshape(B)
    m, r = _sc_gather(cid, initial_mean, initial_std)
    x3 = x.reshape(B, S // 2, 2 * D)
    out = _tc_normalize(x3, m, r)
    return out.reshape(B, S, D)
